# Initial kernel scaffold; baseline (speedup 1.0000x reference)
import jax
import jax.numpy as jnp
from jax.experimental import pallas as pl


def kernel(x, stem_w, stem_b, s0_b0_sc_w, s0_b0_sc_b, s0_b0_conv1_w, s0_b0_conv1_b, s0_b0_conv2_w, s0_b0_conv2_b, s0_b0_conv3_w, s0_b0_conv3_b, s1_b0_sc_w, s1_b0_sc_b, s1_b0_conv1_w, s1_b0_conv1_b, s1_b0_conv2_w, s1_b0_conv2_b, s1_b0_conv3_w, s1_b0_conv3_b, s2_b0_sc_w, s2_b0_sc_b, s2_b0_conv1_w, s2_b0_conv1_b, s2_b0_conv2_w, s2_b0_conv2_b, s2_b0_conv3_w, s2_b0_conv3_b, s3_b0_sc_w, s3_b0_sc_b, s3_b0_conv1_w, s3_b0_conv1_b, s3_b0_conv2_w, s3_b0_conv2_b, s3_b0_conv3_w, s3_b0_conv3_b, lin_w, lin_b):
    raise NotImplementedError("write your pallas kernel here")


# speedup vs baseline: 1.7722x; 1.7722x over previous
import functools

import jax
import jax.numpy as jnp
from jax import lax
from jax.experimental import pallas as pl
from jax.experimental.pallas import tpu as pltpu

_VMEM_LIMIT = 100 * 1024 * 1024

_TAP = {0: (1, 0), 1: (0, 1), 2: (1, 1)}


def _stem_pool_kernel(xc_ref, w_ref, b_ref, o_ref, yp_ref, *, ph, pw):
    c = w_ref.shape[1]
    y = jnp.dot(xc_ref[0].reshape(4 * ph * pw, xc_ref.shape[3]), w_ref[...],
                preferred_element_type=jnp.float32)
    y = jnp.maximum(y + b_ref[...], 0.0)
    yp_ref[...] = jnp.zeros_like(yp_ref)
    yp_ref[:, 1:, 1:, :] = y.reshape(4, ph, pw, c).astype(jnp.bfloat16)
    yp = yp_ref[...]
    out = None
    for ki in range(3):
        a, ri = _TAP[ki]
        for kj in range(3):
            b, ci = _TAP[kj]
            v = lax.slice(yp, (2 * a + b, ri, ci, 0),
                          (2 * a + b + 1, ri + ph, ci + pw, c))
            out = v if out is None else jnp.maximum(out, v)
    o_ref[...] = out


def _stem_maxpool(xc, w, b, *, ph, pw):
    n = xc.shape[0]
    c = w.shape[1]
    kdim = xc.shape[3]
    return pl.pallas_call(
        functools.partial(_stem_pool_kernel, ph=ph, pw=pw),
        out_shape=jax.ShapeDtypeStruct((n, ph, pw, c), jnp.bfloat16),
        grid=(n,),
        in_specs=[
            pl.BlockSpec((1, 4, ph * pw, kdim), lambda i: (i, 0, 0, 0)),
            pl.BlockSpec((kdim, c), lambda i: (0, 0)),
            pl.BlockSpec((1, c), lambda i: (0, 0)),
        ],
        out_specs=pl.BlockSpec((1, ph, pw, c), lambda i: (i, 0, 0, 0)),
        scratch_shapes=[pltpu.VMEM((4, ph + 1, pw + 1, c), jnp.bfloat16)],
        compiler_params=pltpu.CompilerParams(
            dimension_semantics=("parallel",),
            vmem_limit_bytes=_VMEM_LIMIT),
    )(xc, w, b.reshape(1, c).astype(jnp.float32))


def _block_s1_kernel(x_ref, scw_ref, scb_ref, w1_ref, b1_ref, w2_ref, b2_ref,
                     w3_ref, b3_ref, o_ref, y1p_ref, *, h, w, mid, bsz):
    cin = x_ref.shape[3]
    cout = o_ref.shape[3]

    xs = x_ref[...].reshape(bsz * h * w, cin)
    y1 = jnp.dot(xs, w1_ref[...], preferred_element_type=jnp.float32)
    y1 = jnp.maximum(y1 + b1_ref[...], 0.0)
    y1p_ref[...] = jnp.zeros_like(y1p_ref)
    y1p_ref[:, 1:h + 1, 1:w + 1, :] = y1.reshape(bsz, h, w, mid).astype(
        jnp.bfloat16)
    y1p = y1p_ref[...]

    acc = None
    for ki in range(3):
        for kj in range(3):
            v = lax.slice(y1p, (0, ki, kj, 0),
                          (bsz, ki + h, kj + w, mid))
            t = ki * 3 + kj
            p = jnp.dot(v.reshape(bsz * h * w, mid),
                        w2_ref[t * mid:(t + 1) * mid, :],
                        preferred_element_type=jnp.float32)
            acc = p if acc is None else acc + p
    y2 = jnp.maximum(acc + b2_ref[...], 0.0).astype(jnp.bfloat16)

    sc = jnp.dot(xs, scw_ref[...], preferred_element_type=jnp.float32)
    y3 = jnp.dot(y2, w3_ref[...], preferred_element_type=jnp.float32)
    out = jnp.maximum(y3 + b3_ref[...] + sc + scb_ref[...], 0.0)
    o_ref[...] = out.astype(jnp.bfloat16).reshape(bsz, h, w, cout)


def _block_s2_kernel(xph_ref, scw_ref, scb_ref, w1_ref, b1_ref, w2_ref,
                     b2_ref, w3_ref, b3_ref, o_ref, y1p_ref, *, oh, ow, mid,
                     bsz):
    cin = xph_ref.shape[4]
    cout = o_ref.shape[3]

    xs = xph_ref[...].reshape(bsz * 4 * oh * ow, cin)
    y1 = jnp.dot(xs, w1_ref[...], preferred_element_type=jnp.float32)
    y1 = jnp.maximum(y1 + b1_ref[...], 0.0)
    y1p_ref[...] = jnp.zeros_like(y1p_ref)
    y1p_ref[:, :, 1:, 1:, :] = y1.reshape(bsz, 4, oh, ow, mid).astype(
        jnp.bfloat16)
    y1p = y1p_ref[...]

    acc = None
    for ki in range(3):
        a, ri = _TAP[ki]
        for kj in range(3):
            b, ci = _TAP[kj]
            v = lax.slice(y1p, (0, 2 * a + b, ri, ci, 0),
                          (bsz, 2 * a + b + 1, ri + oh, ci + ow, mid))
            t = ki * 3 + kj
            p = jnp.dot(v.reshape(bsz * oh * ow, mid),
                        w2_ref[t * mid:(t + 1) * mid, :],
                        preferred_element_type=jnp.float32)
            acc = p if acc is None else acc + p
    y2 = jnp.maximum(acc + b2_ref[...], 0.0).astype(jnp.bfloat16)

    sc_in = xph_ref[:, 0, :, :, :].reshape(bsz * oh * ow, cin)
    sc = jnp.dot(sc_in, scw_ref[...], preferred_element_type=jnp.float32)
    y3 = jnp.dot(y2, w3_ref[...], preferred_element_type=jnp.float32)
    out = jnp.maximum(y3 + b3_ref[...] + sc + scb_ref[...], 0.0)
    o_ref[...] = out.astype(jnp.bfloat16).reshape(bsz, oh, ow, cout)


def _block(xa, scw, scb, w1, b1, w2, b2, w3, b3, *, stride):
    n, h, w_, cin = xa.shape
    mid = w1.shape[1]
    cout = w3.shape[1]
    oh, ow = h // stride, w_ // stride
    bsz = 1
    while bsz < 8 and n % (bsz * 2) == 0 and bsz * 2 * oh * ow <= 1568:
        bsz *= 2
    grid = (n // bsz,)

    const = lambda i: (0, 0)
    wspecs = [
        pl.BlockSpec((cin, cout), const),
        pl.BlockSpec((1, cout), const),
        pl.BlockSpec((cin, mid), const),
        pl.BlockSpec((1, mid), const),
        pl.BlockSpec((9 * mid, mid), const),
        pl.BlockSpec((1, mid), const),
        pl.BlockSpec((mid, cout), const),
        pl.BlockSpec((1, cout), const),
    ]
    wargs = [scw, scb.reshape(1, cout).astype(jnp.float32),
             w1, b1.reshape(1, mid).astype(jnp.float32),
             w2, b2.reshape(1, mid).astype(jnp.float32),
             w3, b3.reshape(1, cout).astype(jnp.float32)]

    if stride == 1:
        body = functools.partial(_block_s1_kernel, h=h, w=w_, mid=mid,
                                 bsz=bsz)
        args = [xa]
        in_specs = [pl.BlockSpec((bsz, h, w_, cin), lambda i: (i, 0, 0, 0))]
        scratch = pltpu.VMEM((bsz, h + 2, w_ + 2, mid), jnp.bfloat16)
    else:
        xph = jnp.stack([xa[:, a::2, b::2, :]
                         for a in (0, 1) for b in (0, 1)], axis=1)
        body = functools.partial(_block_s2_kernel, oh=oh, ow=ow, mid=mid,
                                 bsz=bsz)
        args = [xph]
        in_specs = [pl.BlockSpec((bsz, 4, oh, ow, cin),
                                 lambda i: (i, 0, 0, 0, 0))]
        scratch = pltpu.VMEM((bsz, 4, oh + 1, ow + 1, mid), jnp.bfloat16)

    return pl.pallas_call(
        body,
        out_shape=jax.ShapeDtypeStruct((n, oh, ow, cout), jnp.bfloat16),
        grid=grid,
        in_specs=in_specs + wspecs,
        out_specs=pl.BlockSpec((bsz, oh, ow, cout), lambda i: (i, 0, 0, 0)),
        scratch_shapes=[scratch],
        compiler_params=pltpu.CompilerParams(
            dimension_semantics=("parallel",),
            vmem_limit_bytes=_VMEM_LIMIT),
    )(*(args + wargs))


def _head_kernel(x_ref, w_ref, b_ref, o_ref):
    m = jnp.mean(x_ref[...].astype(jnp.float32), axis=1)
    out = jnp.dot(m.astype(jnp.bfloat16), w_ref[...],
                  preferred_element_type=jnp.float32)
    o_ref[...] = out + b_ref[...]


def _head(xf, lw, lb):
    n, hw, c = xf.shape
    ncls = lw.shape[1]
    npad = (ncls + 127) // 128 * 128
    wp = jnp.pad(lw, ((0, 0), (0, npad - ncls)))
    bp = jnp.pad(lb.reshape(1, ncls).astype(jnp.float32),
                 ((0, 0), (0, npad - ncls)))
    out = pl.pallas_call(
        _head_kernel,
        out_shape=jax.ShapeDtypeStruct((n, npad), jnp.float32),
        in_specs=[pl.BlockSpec(xf.shape, lambda: (0, 0, 0)),
                  pl.BlockSpec((c, npad), lambda: (0, 0)),
                  pl.BlockSpec((1, npad), lambda: (0, 0))],
        out_specs=pl.BlockSpec((n, npad), lambda: (0, 0)),
        compiler_params=pltpu.CompilerParams(vmem_limit_bytes=_VMEM_LIMIT),
    )(xf, wp, bp)
    return out[:, :ncls]


def kernel(x, stem_w, stem_b,
           s0_b0_sc_w, s0_b0_sc_b, s0_b0_conv1_w, s0_b0_conv1_b,
           s0_b0_conv2_w, s0_b0_conv2_b, s0_b0_conv3_w, s0_b0_conv3_b,
           s1_b0_sc_w, s1_b0_sc_b, s1_b0_conv1_w, s1_b0_conv1_b,
           s1_b0_conv2_w, s1_b0_conv2_b, s1_b0_conv3_w, s1_b0_conv3_b,
           s2_b0_sc_w, s2_b0_sc_b, s2_b0_conv1_w, s2_b0_conv1_b,
           s2_b0_conv2_w, s2_b0_conv2_b, s2_b0_conv3_w, s2_b0_conv3_b,
           s3_b0_sc_w, s3_b0_sc_b, s3_b0_conv1_w, s3_b0_conv1_b,
           s3_b0_conv2_w, s3_b0_conv2_b, s3_b0_conv3_w, s3_b0_conv3_b,
           lin_w, lin_b):
    n, cim, h, w_ = x.shape
    xh = jnp.transpose(x, (0, 2, 3, 1)).astype(jnp.bfloat16)
    xp = jnp.pad(xh, ((0, 0), (3, 3), (3, 3), (0, 0)))
    oh, ow = h // 2, w_ // 2
    ph, pw = oh // 2, ow // 2
    phases = []
    for a in (0, 1):
        for b in (0, 1):
            cols = []
            for ki in range(7):
                for kj in range(7):
                    cols.append(
                        xp[:, ki + 2 * a:ki + 2 * a + 4 * (ph - 1) + 1:4,
                           kj + 2 * b:kj + 2 * b + 4 * (pw - 1) + 1:4, :])
            phases.append(jnp.concatenate(cols, axis=-1))
    patches = jnp.stack(phases, axis=1).reshape(n, 4, ph * pw, 49 * cim)

    xa = _stem_maxpool(patches, stem_w, stem_b, ph=ph, pw=pw)

    xa = _block(xa, s0_b0_sc_w, s0_b0_sc_b, s0_b0_conv1_w, s0_b0_conv1_b,
                s0_b0_conv2_w, s0_b0_conv2_b, s0_b0_conv3_w, s0_b0_conv3_b,
                stride=1)
    xa = _block(xa, s1_b0_sc_w, s1_b0_sc_b, s1_b0_conv1_w, s1_b0_conv1_b,
                s1_b0_conv2_w, s1_b0_conv2_b, s1_b0_conv3_w, s1_b0_conv3_b,
                stride=2)
    xa = _block(xa, s2_b0_sc_w, s2_b0_sc_b, s2_b0_conv1_w, s2_b0_conv1_b,
                s2_b0_conv2_w, s2_b0_conv2_b, s2_b0_conv3_w, s2_b0_conv3_b,
                stride=2)
    xa = _block(xa, s3_b0_sc_w, s3_b0_sc_b, s3_b0_conv1_w, s3_b0_conv1_b,
                s3_b0_conv2_w, s3_b0_conv2_b, s3_b0_conv3_w, s3_b0_conv3_b,
                stride=2)

    nb, fh, fw, fc = xa.shape
    return _head(xa.reshape(nb, fh * fw, fc), lin_w, lin_b)


# speedup vs baseline: 4.0825x; 2.3036x over previous
import functools

import jax
import jax.numpy as jnp
import numpy as np
from jax import lax
from jax.experimental import pallas as pl
from jax.experimental.pallas import tpu as pltpu

_VMEM_LIMIT = 100 * 1024 * 1024


def _stem_phase_matrix(cin):
    p = np.zeros((4, 9 * 16 * cin, 49 * cin), np.float32)
    for pa in range(2):
        for pb in range(2):
            ph = pa * 2 + pb
            for ki in range(7):
                sr, qr = divmod(2 * pa + ki, 4)
                for kj in range(7):
                    sc, qc = divmod(2 * pb + kj, 4)
                    t = sr * 3 + sc
                    for c in range(cin):
                        p[ph, (t * 16 + qr * 4 + qc) * cin + c,
                          (ki * 7 + kj) * cin + c] = 1.0
    return p


_STEM_P3 = _stem_phase_matrix(3)

_TAP = {0: (1, 0), 1: (0, 1), 2: (1, 1)}


def _stem_pool_kernel(xq_ref, w_ref, b_ref, o_ref, yp_ref, *, ph, pw):
    c = o_ref.shape[3]
    ck = xq_ref.shape[3]
    xq = xq_ref[0]
    vs = [lax.slice(xq, (sr, sc, 0),
                    (sr + ph, sc + pw, ck)).reshape(ph * pw, ck)
          for sr in range(3) for sc in range(3)]
    yp_ref[...] = jnp.zeros_like(yp_ref)
    for p in range(4):
        acc = None
        for t in range(9):
            r0 = (p * 9 + t) * ck
            pp = jnp.dot(vs[t], w_ref[r0:r0 + ck, :],
                         preferred_element_type=jnp.float32)
            acc = pp if acc is None else acc + pp
        y = jnp.maximum(acc + b_ref[...], 0.0)
        yp_ref[p, 1:, 1:, :] = y.reshape(ph, pw, c).astype(jnp.bfloat16)
    yp = yp_ref[...]
    out = None
    for ki in range(3):
        a, ri = _TAP[ki]
        for kj in range(3):
            b, ci = _TAP[kj]
            v = lax.slice(yp, (2 * a + b, ri, ci, 0),
                          (2 * a + b + 1, ri + ph, ci + pw, c))
            out = v if out is None else jnp.maximum(out, v)
    o_ref[...] = out


def _stem_maxpool(xq, w4, b, *, ph, pw):
    n = xq.shape[0]
    c = w4.shape[1]
    kdim = xq.shape[3]
    return pl.pallas_call(
        functools.partial(_stem_pool_kernel, ph=ph, pw=pw),
        out_shape=jax.ShapeDtypeStruct((n, ph, pw, c), jnp.bfloat16),
        grid=(n,),
        in_specs=[
            pl.BlockSpec((1, ph + 2, pw + 2, kdim), lambda i: (i, 0, 0, 0)),
            pl.BlockSpec((36 * kdim, c), lambda i: (0, 0)),
            pl.BlockSpec((1, c), lambda i: (0, 0)),
        ],
        out_specs=pl.BlockSpec((1, ph, pw, c), lambda i: (i, 0, 0, 0)),
        scratch_shapes=[pltpu.VMEM((4, ph + 1, pw + 1, c), jnp.bfloat16)],
        compiler_params=pltpu.CompilerParams(
            dimension_semantics=("parallel",),
            vmem_limit_bytes=_VMEM_LIMIT),
    )(xq, w4, b.reshape(1, c).astype(jnp.float32))


def _block_s1_kernel(x_ref, scw_ref, scb_ref, w1_ref, b1_ref, w2_ref, b2_ref,
                     w3_ref, b3_ref, o_ref, y1p_ref, *, h, w, mid, bsz):
    cin = x_ref.shape[3]
    cout = o_ref.shape[3]

    xs = x_ref[...].reshape(bsz * h * w, cin)
    y1 = jnp.dot(xs, w1_ref[...], preferred_element_type=jnp.float32)
    y1 = jnp.maximum(y1 + b1_ref[...], 0.0)
    y1p_ref[...] = jnp.zeros_like(y1p_ref)
    y1p_ref[:, 1:h + 1, 1:w + 1, :] = y1.reshape(bsz, h, w, mid).astype(
        jnp.bfloat16)
    y1p = y1p_ref[...]

    acc = None
    for ki in range(3):
        for kj in range(3):
            v = lax.slice(y1p, (0, ki, kj, 0),
                          (bsz, ki + h, kj + w, mid))
            t = ki * 3 + kj
            p = jnp.dot(v.reshape(bsz * h * w, mid),
                        w2_ref[t * mid:(t + 1) * mid, :],
                        preferred_element_type=jnp.float32)
            acc = p if acc is None else acc + p
    y2 = jnp.maximum(acc + b2_ref[...], 0.0).astype(jnp.bfloat16)

    sc = jnp.dot(xs, scw_ref[...], preferred_element_type=jnp.float32)
    y3 = jnp.dot(y2, w3_ref[...], preferred_element_type=jnp.float32)
    out = jnp.maximum(y3 + b3_ref[...] + sc + scb_ref[...], 0.0)
    o_ref[...] = out.astype(jnp.bfloat16).reshape(bsz, h, w, cout)


def _block_s2_kernel(xph_ref, scw_ref, scb_ref, w1_ref, b1_ref, w2_ref,
                     b2_ref, w3_ref, b3_ref, o_ref, y1p_ref, *, oh, ow, mid,
                     bsz):
    cin = xph_ref.shape[4]
    cout = o_ref.shape[3]

    xs = xph_ref[...].reshape(bsz * 4 * oh * ow, cin)
    y1 = jnp.dot(xs, w1_ref[...], preferred_element_type=jnp.float32)
    y1 = jnp.maximum(y1 + b1_ref[...], 0.0)
    y1p_ref[...] = jnp.zeros_like(y1p_ref)
    y1p_ref[:, :, 1:, 1:, :] = y1.reshape(bsz, 4, oh, ow, mid).astype(
        jnp.bfloat16)
    y1p = y1p_ref[...]

    acc = None
    for ki in range(3):
        a, ri = _TAP[ki]
        for kj in range(3):
            b, ci = _TAP[kj]
            v = lax.slice(y1p, (0, 2 * a + b, ri, ci, 0),
                          (bsz, 2 * a + b + 1, ri + oh, ci + ow, mid))
            t = ki * 3 + kj
            p = jnp.dot(v.reshape(bsz * oh * ow, mid),
                        w2_ref[t * mid:(t + 1) * mid, :],
                        preferred_element_type=jnp.float32)
            acc = p if acc is None else acc + p
    y2 = jnp.maximum(acc + b2_ref[...], 0.0).astype(jnp.bfloat16)

    sc_in = xph_ref[:, 0, :, :, :].reshape(bsz * oh * ow, cin)
    sc = jnp.dot(sc_in, scw_ref[...], preferred_element_type=jnp.float32)
    y3 = jnp.dot(y2, w3_ref[...], preferred_element_type=jnp.float32)
    out = jnp.maximum(y3 + b3_ref[...] + sc + scb_ref[...], 0.0)
    o_ref[...] = out.astype(jnp.bfloat16).reshape(bsz, oh, ow, cout)


def _block(xa, scw, scb, w1, b1, w2, b2, w3, b3, *, stride):
    n, h, w_, cin = xa.shape
    mid = w1.shape[1]
    cout = w3.shape[1]
    oh, ow = h // stride, w_ // stride
    bsz = 1
    while bsz < 8 and n % (bsz * 2) == 0 and bsz * 2 * oh * ow <= 1568:
        bsz *= 2
    grid = (n // bsz,)

    const = lambda i: (0, 0)
    wspecs = [
        pl.BlockSpec((cin, cout), const),
        pl.BlockSpec((1, cout), const),
        pl.BlockSpec((cin, mid), const),
        pl.BlockSpec((1, mid), const),
        pl.BlockSpec((9 * mid, mid), const),
        pl.BlockSpec((1, mid), const),
        pl.BlockSpec((mid, cout), const),
        pl.BlockSpec((1, cout), const),
    ]
    wargs = [scw, scb.reshape(1, cout).astype(jnp.float32),
             w1, b1.reshape(1, mid).astype(jnp.float32),
             w2, b2.reshape(1, mid).astype(jnp.float32),
             w3, b3.reshape(1, cout).astype(jnp.float32)]

    if stride == 1:
        body = functools.partial(_block_s1_kernel, h=h, w=w_, mid=mid,
                                 bsz=bsz)
        args = [xa]
        in_specs = [pl.BlockSpec((bsz, h, w_, cin), lambda i: (i, 0, 0, 0))]
        scratch = pltpu.VMEM((bsz, h + 2, w_ + 2, mid), jnp.bfloat16)
    else:
        xph = jnp.stack([xa[:, a::2, b::2, :]
                         for a in (0, 1) for b in (0, 1)], axis=1)
        body = functools.partial(_block_s2_kernel, oh=oh, ow=ow, mid=mid,
                                 bsz=bsz)
        args = [xph]
        in_specs = [pl.BlockSpec((bsz, 4, oh, ow, cin),
                                 lambda i: (i, 0, 0, 0, 0))]
        scratch = pltpu.VMEM((bsz, 4, oh + 1, ow + 1, mid), jnp.bfloat16)

    return pl.pallas_call(
        body,
        out_shape=jax.ShapeDtypeStruct((n, oh, ow, cout), jnp.bfloat16),
        grid=grid,
        in_specs=in_specs + wspecs,
        out_specs=pl.BlockSpec((bsz, oh, ow, cout), lambda i: (i, 0, 0, 0)),
        scratch_shapes=[scratch],
        compiler_params=pltpu.CompilerParams(
            dimension_semantics=("parallel",),
            vmem_limit_bytes=_VMEM_LIMIT),
    )(*(args + wargs))


def _head_kernel(x_ref, w_ref, b_ref, o_ref):
    m = jnp.mean(x_ref[...].astype(jnp.float32), axis=1)
    out = jnp.dot(m.astype(jnp.bfloat16), w_ref[...],
                  preferred_element_type=jnp.float32)
    o_ref[...] = out + b_ref[...]


def _head(xf, lw, lb):
    n, hw, c = xf.shape
    ncls = lw.shape[1]
    npad = (ncls + 127) // 128 * 128
    wp = jnp.pad(lw, ((0, 0), (0, npad - ncls)))
    bp = jnp.pad(lb.reshape(1, ncls).astype(jnp.float32),
                 ((0, 0), (0, npad - ncls)))
    out = pl.pallas_call(
        _head_kernel,
        out_shape=jax.ShapeDtypeStruct((n, npad), jnp.float32),
        in_specs=[pl.BlockSpec(xf.shape, lambda: (0, 0, 0)),
                  pl.BlockSpec((c, npad), lambda: (0, 0)),
                  pl.BlockSpec((1, npad), lambda: (0, 0))],
        out_specs=pl.BlockSpec((n, npad), lambda: (0, 0)),
        compiler_params=pltpu.CompilerParams(vmem_limit_bytes=_VMEM_LIMIT),
    )(xf, wp, bp)
    return out[:, :ncls]


def kernel(x, stem_w, stem_b,
           s0_b0_sc_w, s0_b0_sc_b, s0_b0_conv1_w, s0_b0_conv1_b,
           s0_b0_conv2_w, s0_b0_conv2_b, s0_b0_conv3_w, s0_b0_conv3_b,
           s1_b0_sc_w, s1_b0_sc_b, s1_b0_conv1_w, s1_b0_conv1_b,
           s1_b0_conv2_w, s1_b0_conv2_b, s1_b0_conv3_w, s1_b0_conv3_b,
           s2_b0_sc_w, s2_b0_sc_b, s2_b0_conv1_w, s2_b0_conv1_b,
           s2_b0_conv2_w, s2_b0_conv2_b, s2_b0_conv3_w, s2_b0_conv3_b,
           s3_b0_sc_w, s3_b0_sc_b, s3_b0_conv1_w, s3_b0_conv1_b,
           s3_b0_conv2_w, s3_b0_conv2_b, s3_b0_conv3_w, s3_b0_conv3_b,
           lin_w, lin_b):
    n, cim, h, w_ = x.shape
    xh = jnp.transpose(x, (0, 2, 3, 1)).astype(jnp.bfloat16)
    xp = jnp.pad(xh, ((0, 0), (3, 5), (3, 5), (0, 0)))
    oh, ow = h // 2, w_ // 2
    ph, pw = oh // 2, ow // 2
    xq = xp.reshape(n, ph + 2, 4, pw + 2, 4, cim).transpose(
        0, 1, 3, 2, 4, 5).reshape(n, ph + 2, pw + 2, 16 * cim)
    w4 = jnp.matmul(jnp.asarray(_STEM_P3, jnp.bfloat16),
                    stem_w.astype(jnp.bfloat16)).reshape(-1, stem_w.shape[1])

    xa = _stem_maxpool(xq, w4, stem_b, ph=ph, pw=pw)

    xa = _block(xa, s0_b0_sc_w, s0_b0_sc_b, s0_b0_conv1_w, s0_b0_conv1_b,
                s0_b0_conv2_w, s0_b0_conv2_b, s0_b0_conv3_w, s0_b0_conv3_b,
                stride=1)
    xa = _block(xa, s1_b0_sc_w, s1_b0_sc_b, s1_b0_conv1_w, s1_b0_conv1_b,
                s1_b0_conv2_w, s1_b0_conv2_b, s1_b0_conv3_w, s1_b0_conv3_b,
                stride=2)
    xa = _block(xa, s2_b0_sc_w, s2_b0_sc_b, s2_b0_conv1_w, s2_b0_conv1_b,
                s2_b0_conv2_w, s2_b0_conv2_b, s2_b0_conv3_w, s2_b0_conv3_b,
                stride=2)
    xa = _block(xa, s3_b0_sc_w, s3_b0_sc_b, s3_b0_conv1_w, s3_b0_conv1_b,
                s3_b0_conv2_w, s3_b0_conv2_b, s3_b0_conv3_w, s3_b0_conv3_b,
                stride=2)

    nb, fh, fw, fc = xa.shape
    return _head(xa.reshape(nb, fh * fw, fc), lin_w, lin_b)


# speedup vs baseline: 14.3729x; 3.5206x over previous
import functools

import jax
import jax.numpy as jnp
import numpy as np
from jax import lax
from jax.experimental import pallas as pl
from jax.experimental.pallas import tpu as pltpu

_VMEM_LIMIT = 100 * 1024 * 1024


def _stem_phase_matrix(cin):
    p = np.zeros((4, 9 * 16 * cin, 49 * cin), np.float32)
    for pa in range(2):
        for pb in range(2):
            ph = pa * 2 + pb
            for ki in range(7):
                sr, qr = divmod(2 * pa + ki, 4)
                for kj in range(7):
                    sc, qc = divmod(2 * pb + kj, 4)
                    t = sr * 3 + sc
                    for c in range(cin):
                        p[ph, (t * 16 + qr * 4 + qc) * cin + c,
                          (ki * 7 + kj) * cin + c] = 1.0
    return p


_STEM_P3 = _stem_phase_matrix(3)

_TAP = {0: (1, 0), 1: (0, 1), 2: (1, 1)}


def _stem_pool_kernel(xq_ref, w_ref, b_ref, o_ref, yp_ref, *, ph, pw):
    c = o_ref.shape[3]
    ck = xq_ref.shape[3]
    xq = xq_ref[0]
    vs = [lax.slice(xq, (sr, sc, 0),
                    (sr + ph, sc + pw, ck)).reshape(ph * pw, ck)
          for sr in range(3) for sc in range(3)]
    yp_ref[...] = jnp.zeros_like(yp_ref)
    for p in range(4):
        acc = None
        for t in range(9):
            r0 = (p * 9 + t) * ck
            pp = jnp.dot(vs[t], w_ref[r0:r0 + ck, :],
                         preferred_element_type=jnp.float32)
            acc = pp if acc is None else acc + pp
        y = jnp.maximum(acc + b_ref[...], 0.0)
        yp_ref[p, 1:, 1:, :] = y.reshape(ph, pw, c).astype(jnp.bfloat16)
    yp = yp_ref[...]
    out = None
    for ki in range(3):
        a, ri = _TAP[ki]
        for kj in range(3):
            b, ci = _TAP[kj]
            v = lax.slice(yp, (2 * a + b, ri, ci, 0),
                          (2 * a + b + 1, ri + ph, ci + pw, c))
            out = v if out is None else jnp.maximum(out, v)
    o_ref[...] = out


def _stem_maxpool(xq, w4, b, *, ph, pw):
    n = xq.shape[0]
    c = w4.shape[1]
    kdim = xq.shape[3]
    return pl.pallas_call(
        functools.partial(_stem_pool_kernel, ph=ph, pw=pw),
        out_shape=jax.ShapeDtypeStruct((n, ph, pw, c), jnp.bfloat16),
        grid=(n,),
        in_specs=[
            pl.BlockSpec((1, ph + 2, pw + 2, kdim), lambda i: (i, 0, 0, 0)),
            pl.BlockSpec((36 * kdim, c), lambda i: (0, 0)),
            pl.BlockSpec((1, c), lambda i: (0, 0)),
        ],
        out_specs=pl.BlockSpec((1, ph, pw, c), lambda i: (i, 0, 0, 0)),
        scratch_shapes=[pltpu.VMEM((4, ph + 1, pw + 1, c), jnp.bfloat16)],
        compiler_params=pltpu.CompilerParams(
            dimension_semantics=("parallel",),
            vmem_limit_bytes=_VMEM_LIMIT),
    )(xq, w4, b.reshape(1, c).astype(jnp.float32))


def _block_s1_kernel(x_ref, scw_ref, scb_ref, w1_ref, b1_ref, w2_ref, b2_ref,
                     w3_ref, b3_ref, o_ref, y1p_ref, *, h, w, mid, bsz):
    cin = x_ref.shape[3]
    cout = o_ref.shape[3]

    xs = x_ref[...].reshape(bsz * h * w, cin)
    y1 = jnp.dot(xs, w1_ref[...], preferred_element_type=jnp.float32)
    y1 = jnp.maximum(y1 + b1_ref[...], 0.0)
    y1p_ref[...] = jnp.zeros_like(y1p_ref)
    y1p_ref[:, 1:h + 1, 1:w + 1, :] = y1.reshape(bsz, h, w, mid).astype(
        jnp.bfloat16)
    y1p = y1p_ref[...]

    acc = None
    for ki in range(3):
        for kj in range(3):
            v = lax.slice(y1p, (0, ki, kj, 0),
                          (bsz, ki + h, kj + w, mid))
            t = ki * 3 + kj
            p = jnp.dot(v.reshape(bsz * h * w, mid),
                        w2_ref[t * mid:(t + 1) * mid, :],
                        preferred_element_type=jnp.float32)
            acc = p if acc is None else acc + p
    y2 = jnp.maximum(acc + b2_ref[...], 0.0).astype(jnp.bfloat16)

    sc = jnp.dot(xs, scw_ref[...], preferred_element_type=jnp.float32)
    y3 = jnp.dot(y2, w3_ref[...], preferred_element_type=jnp.float32)
    out = jnp.maximum(y3 + b3_ref[...] + sc + scb_ref[...], 0.0)
    o_ref[...] = out.astype(jnp.bfloat16).reshape(bsz, h, w, cout)


def _block_s2_kernel(xph_ref, scw_ref, scb_ref, w1_ref, b1_ref, w2_ref,
                     b2_ref, w3_ref, b3_ref, o_ref, y1p_ref, *, oh, ow, mid,
                     bsz):
    cin = xph_ref.shape[4]
    cout = o_ref.shape[3]

    xs = xph_ref[...].reshape(bsz * 4 * oh * ow, cin)
    y1 = jnp.dot(xs, w1_ref[...], preferred_element_type=jnp.float32)
    y1 = jnp.maximum(y1 + b1_ref[...], 0.0)
    y1p_ref[...] = jnp.zeros_like(y1p_ref)
    y1p_ref[:, :, 1:, 1:, :] = y1.reshape(bsz, 4, oh, ow, mid).astype(
        jnp.bfloat16)
    y1p = y1p_ref[...]

    acc = None
    for ki in range(3):
        a, ri = _TAP[ki]
        for kj in range(3):
            b, ci = _TAP[kj]
            v = lax.slice(y1p, (0, 2 * a + b, ri, ci, 0),
                          (bsz, 2 * a + b + 1, ri + oh, ci + ow, mid))
            t = ki * 3 + kj
            p = jnp.dot(v.reshape(bsz * oh * ow, mid),
                        w2_ref[t * mid:(t + 1) * mid, :],
                        preferred_element_type=jnp.float32)
            acc = p if acc is None else acc + p
    y2 = jnp.maximum(acc + b2_ref[...], 0.0).astype(jnp.bfloat16)

    sc_in = xph_ref[:, 0, :, :, :].reshape(bsz * oh * ow, cin)
    sc = jnp.dot(sc_in, scw_ref[...], preferred_element_type=jnp.float32)
    y3 = jnp.dot(y2, w3_ref[...], preferred_element_type=jnp.float32)
    out = jnp.maximum(y3 + b3_ref[...] + sc + scb_ref[...], 0.0)
    o_ref[...] = out.astype(jnp.bfloat16).reshape(bsz, oh, ow, cout)


def _block(xa, scw, scb, w1, b1, w2, b2, w3, b3, *, stride):
    n, h, w_, cin = xa.shape
    mid = w1.shape[1]
    cout = w3.shape[1]
    oh, ow = h // stride, w_ // stride
    bsz = 1
    while bsz < 8 and n % (bsz * 2) == 0 and bsz * 2 * oh * ow <= 1568:
        bsz *= 2
    grid = (n // bsz,)

    const = lambda i: (0, 0)
    wspecs = [
        pl.BlockSpec((cin, cout), const),
        pl.BlockSpec((1, cout), const),
        pl.BlockSpec((cin, mid), const),
        pl.BlockSpec((1, mid), const),
        pl.BlockSpec((9 * mid, mid), const),
        pl.BlockSpec((1, mid), const),
        pl.BlockSpec((mid, cout), const),
        pl.BlockSpec((1, cout), const),
    ]
    wargs = [scw, scb.reshape(1, cout).astype(jnp.float32),
             w1, b1.reshape(1, mid).astype(jnp.float32),
             w2, b2.reshape(1, mid).astype(jnp.float32),
             w3, b3.reshape(1, cout).astype(jnp.float32)]

    if stride == 1:
        body = functools.partial(_block_s1_kernel, h=h, w=w_, mid=mid,
                                 bsz=bsz)
        args = [xa]
        in_specs = [pl.BlockSpec((bsz, h, w_, cin), lambda i: (i, 0, 0, 0))]
        scratch = pltpu.VMEM((bsz, h + 2, w_ + 2, mid), jnp.bfloat16)
    else:
        xph = xa.reshape(n, oh, 2, ow, 2, cin).transpose(
            0, 2, 4, 1, 3, 5).reshape(n, 4, oh, ow, cin)
        body = functools.partial(_block_s2_kernel, oh=oh, ow=ow, mid=mid,
                                 bsz=bsz)
        args = [xph]
        in_specs = [pl.BlockSpec((bsz, 4, oh, ow, cin),
                                 lambda i: (i, 0, 0, 0, 0))]
        scratch = pltpu.VMEM((bsz, 4, oh + 1, ow + 1, mid), jnp.bfloat16)

    return pl.pallas_call(
        body,
        out_shape=jax.ShapeDtypeStruct((n, oh, ow, cout), jnp.bfloat16),
        grid=grid,
        in_specs=in_specs + wspecs,
        out_specs=pl.BlockSpec((bsz, oh, ow, cout), lambda i: (i, 0, 0, 0)),
        scratch_shapes=[scratch],
        compiler_params=pltpu.CompilerParams(
            dimension_semantics=("parallel",),
            vmem_limit_bytes=_VMEM_LIMIT),
    )(*(args + wargs))


def _head_kernel(x_ref, w_ref, b_ref, o_ref):
    m = jnp.mean(x_ref[...].astype(jnp.float32), axis=1)
    out = jnp.dot(m.astype(jnp.bfloat16), w_ref[...],
                  preferred_element_type=jnp.float32)
    o_ref[...] = out + b_ref[...]


def _head(xf, lw, lb):
    n, hw, c = xf.shape
    ncls = lw.shape[1]
    npad = (ncls + 127) // 128 * 128
    wp = jnp.pad(lw, ((0, 0), (0, npad - ncls)))
    bp = jnp.pad(lb.reshape(1, ncls).astype(jnp.float32),
                 ((0, 0), (0, npad - ncls)))
    out = pl.pallas_call(
        _head_kernel,
        out_shape=jax.ShapeDtypeStruct((n, npad), jnp.float32),
        in_specs=[pl.BlockSpec(xf.shape, lambda: (0, 0, 0)),
                  pl.BlockSpec((c, npad), lambda: (0, 0)),
                  pl.BlockSpec((1, npad), lambda: (0, 0))],
        out_specs=pl.BlockSpec((n, npad), lambda: (0, 0)),
        compiler_params=pltpu.CompilerParams(vmem_limit_bytes=_VMEM_LIMIT),
    )(xf, wp, bp)
    return out[:, :ncls]


def kernel(x, stem_w, stem_b,
           s0_b0_sc_w, s0_b0_sc_b, s0_b0_conv1_w, s0_b0_conv1_b,
           s0_b0_conv2_w, s0_b0_conv2_b, s0_b0_conv3_w, s0_b0_conv3_b,
           s1_b0_sc_w, s1_b0_sc_b, s1_b0_conv1_w, s1_b0_conv1_b,
           s1_b0_conv2_w, s1_b0_conv2_b, s1_b0_conv3_w, s1_b0_conv3_b,
           s2_b0_sc_w, s2_b0_sc_b, s2_b0_conv1_w, s2_b0_conv1_b,
           s2_b0_conv2_w, s2_b0_conv2_b, s2_b0_conv3_w, s2_b0_conv3_b,
           s3_b0_sc_w, s3_b0_sc_b, s3_b0_conv1_w, s3_b0_conv1_b,
           s3_b0_conv2_w, s3_b0_conv2_b, s3_b0_conv3_w, s3_b0_conv3_b,
           lin_w, lin_b):
    n, cim, h, w_ = x.shape
    xh = jnp.transpose(x, (0, 2, 3, 1)).astype(jnp.bfloat16)
    xp = jnp.pad(xh, ((0, 0), (3, 5), (3, 5), (0, 0)))
    oh, ow = h // 2, w_ // 2
    ph, pw = oh // 2, ow // 2
    xq = xp.reshape(n, ph + 2, 4, pw + 2, 4, cim).transpose(
        0, 1, 3, 2, 4, 5).reshape(n, ph + 2, pw + 2, 16 * cim)
    w4 = jnp.matmul(jnp.asarray(_STEM_P3, jnp.bfloat16),
                    stem_w.astype(jnp.bfloat16)).reshape(-1, stem_w.shape[1])

    xa = _stem_maxpool(xq, w4, stem_b, ph=ph, pw=pw)

    xa = _block(xa, s0_b0_sc_w, s0_b0_sc_b, s0_b0_conv1_w, s0_b0_conv1_b,
                s0_b0_conv2_w, s0_b0_conv2_b, s0_b0_conv3_w, s0_b0_conv3_b,
                stride=1)
    xa = _block(xa, s1_b0_sc_w, s1_b0_sc_b, s1_b0_conv1_w, s1_b0_conv1_b,
                s1_b0_conv2_w, s1_b0_conv2_b, s1_b0_conv3_w, s1_b0_conv3_b,
                stride=2)
    xa = _block(xa, s2_b0_sc_w, s2_b0_sc_b, s2_b0_conv1_w, s2_b0_conv1_b,
                s2_b0_conv2_w, s2_b0_conv2_b, s2_b0_conv3_w, s2_b0_conv3_b,
                stride=2)
    xa = _block(xa, s3_b0_sc_w, s3_b0_sc_b, s3_b0_conv1_w, s3_b0_conv1_b,
                s3_b0_conv2_w, s3_b0_conv2_b, s3_b0_conv3_w, s3_b0_conv3_b,
                stride=2)

    nb, fh, fw, fc = xa.shape
    return _head(xa.reshape(nb, fh * fw, fc), lin_w, lin_b)
